# baseline (speedup 1.0000x reference)
import jax
import jax.numpy as jnp
from jax.experimental import pallas as pl
from jax.experimental.pallas import tpu as pltpu


def _support_kernel(x_ref, w_ref, o_ref):
    o_ref[...] = jnp.dot(x_ref[...], w_ref[...],
                         preferred_element_type=jnp.float32)


def _main_kernel(adj_ref, s_ref, t_ref, bgc_ref, W00_ref, b00_ref,
                 W10_ref, b10_ref, w01_ref, b01_ref, w11_ref, b11_ref,
                 wpp_ref, bpp_ref, rep_ref, y_ref, p1_ref):
    acc = jnp.dot(adj_ref[...], s_ref[...],
                  preferred_element_type=jnp.float32)
    rep = jnp.maximum(acc + bgc_ref[...], 0.0)
    rep_ref[...] = rep
    y00 = jnp.maximum(
        jnp.dot(rep, W00_ref[...], preferred_element_type=jnp.float32)
        + b00_ref[...], 0.0)
    y10 = jnp.maximum(
        jnp.dot(rep, W10_ref[...], preferred_element_type=jnp.float32)
        + b10_ref[...], 0.0)
    y0 = jnp.dot(y00, w01_ref[...],
                 preferred_element_type=jnp.float32) + b01_ref[...]
    y1 = jnp.dot(y10, w11_ref[...],
                 preferred_element_type=jnp.float32) + b11_ref[...]
    y_ref[...] = jnp.where(t_ref[...] > 0, y1, y0)
    p1_ref[...] = jax.nn.sigmoid(
        jnp.dot(rep, wpp_ref[...], preferred_element_type=jnp.float32)
        + bpp_ref[...])


def kernel(x, adj, t, W_gc, b_gc, W00, b00, W10, b10, w01, b01, w11,
           b11, wpp, bpp):
    N, F = x.shape
    H = W_gc.shape[1]
    BM = 200

    support = pl.pallas_call(
        _support_kernel,
        out_shape=jax.ShapeDtypeStruct((N, H), jnp.float32),
    )(x, W_gc)

    t2 = t.reshape(N, 1)
    bgc2 = b_gc.reshape(1, H)
    b002 = b00.reshape(1, H)
    b102 = b10.reshape(1, H)
    b012 = b01.reshape(1, 1)
    b112 = b11.reshape(1, 1)
    bpp2 = bpp.reshape(1, 1)

    full = lambda shape: pl.BlockSpec(shape, lambda i: (0, 0))
    grid = (N // BM,)
    rep, y, p1 = pl.pallas_call(
        _main_kernel,
        grid=grid,
        in_specs=[
            pl.BlockSpec((BM, N), lambda i: (i, 0)),
            full((N, H)),
            pl.BlockSpec((BM, 1), lambda i: (i, 0)),
            full((1, H)),
            full((H, H)),
            full((1, H)),
            full((H, H)),
            full((1, H)),
            full((H, 1)),
            full((1, 1)),
            full((H, 1)),
            full((1, 1)),
            full((H, 1)),
            full((1, 1)),
        ],
        out_specs=[
            pl.BlockSpec((BM, H), lambda i: (i, 0)),
            pl.BlockSpec((BM, 1), lambda i: (i, 0)),
            pl.BlockSpec((BM, 1), lambda i: (i, 0)),
        ],
        out_shape=[
            jax.ShapeDtypeStruct((N, H), jnp.float32),
            jax.ShapeDtypeStruct((N, 1), jnp.float32),
            jax.ShapeDtypeStruct((N, 1), jnp.float32),
        ],
        compiler_params=pltpu.CompilerParams(
            dimension_semantics=("arbitrary",)),
    )(adj, support, t2, bgc2, W00, b002, W10, b102,
      w01, b012, w11, b112, wpp, bpp2)

    return y.reshape(-1), rep, p1.reshape(-1)


# speedup vs baseline: 1.0239x; 1.0239x over previous
import jax
import jax.numpy as jnp
from jax.experimental import pallas as pl
from jax.experimental.pallas import tpu as pltpu


def _support_kernel(x_ref, w_ref, o_ref):
    o_ref[...] = jnp.dot(x_ref[...], w_ref[...],
                         preferred_element_type=jnp.float32
                         ).astype(jnp.bfloat16)


def _main_kernel(adj_ref, s_ref, t_ref, bgc_ref, W00_ref, b00_ref,
                 W10_ref, b10_ref, w01_ref, b01_ref, w11_ref, b11_ref,
                 wpp_ref, bpp_ref, rep_ref, y_ref, p1_ref):
    acc = jnp.dot(adj_ref[...].astype(jnp.bfloat16), s_ref[...],
                  preferred_element_type=jnp.float32)
    rep = jnp.maximum(acc + bgc_ref[...], 0.0)
    rep_ref[...] = rep
    y00 = jnp.maximum(
        jnp.dot(rep, W00_ref[...], preferred_element_type=jnp.float32)
        + b00_ref[...], 0.0)
    y10 = jnp.maximum(
        jnp.dot(rep, W10_ref[...], preferred_element_type=jnp.float32)
        + b10_ref[...], 0.0)
    y0 = jnp.dot(y00, w01_ref[...],
                 preferred_element_type=jnp.float32) + b01_ref[...]
    y1 = jnp.dot(y10, w11_ref[...],
                 preferred_element_type=jnp.float32) + b11_ref[...]
    y_ref[...] = jnp.where(t_ref[...] > 0, y1, y0)
    p1_ref[...] = jax.nn.sigmoid(
        jnp.dot(rep, wpp_ref[...], preferred_element_type=jnp.float32)
        + bpp_ref[...])


def kernel(x, adj, t, W_gc, b_gc, W00, b00, W10, b10, w01, b01, w11,
           b11, wpp, bpp):
    N, F = x.shape
    H = W_gc.shape[1]
    BM = 200

    support = pl.pallas_call(
        _support_kernel,
        out_shape=jax.ShapeDtypeStruct((N, H), jnp.bfloat16),
    )(x, W_gc)

    t2 = t.reshape(N, 1)
    bgc2 = b_gc.reshape(1, H)
    b002 = b00.reshape(1, H)
    b102 = b10.reshape(1, H)
    b012 = b01.reshape(1, 1)
    b112 = b11.reshape(1, 1)
    bpp2 = bpp.reshape(1, 1)

    full = lambda shape: pl.BlockSpec(shape, lambda i: (0, 0))
    grid = (N // BM,)
    rep, y, p1 = pl.pallas_call(
        _main_kernel,
        grid=grid,
        in_specs=[
            pl.BlockSpec((BM, N), lambda i: (i, 0)),
            full((N, H)),
            pl.BlockSpec((BM, 1), lambda i: (i, 0)),
            full((1, H)),
            full((H, H)),
            full((1, H)),
            full((H, H)),
            full((1, H)),
            full((H, 1)),
            full((1, 1)),
            full((H, 1)),
            full((1, 1)),
            full((H, 1)),
            full((1, 1)),
        ],
        out_specs=[
            pl.BlockSpec((BM, H), lambda i: (i, 0)),
            pl.BlockSpec((BM, 1), lambda i: (i, 0)),
            pl.BlockSpec((BM, 1), lambda i: (i, 0)),
        ],
        out_shape=[
            jax.ShapeDtypeStruct((N, H), jnp.float32),
            jax.ShapeDtypeStruct((N, 1), jnp.float32),
            jax.ShapeDtypeStruct((N, 1), jnp.float32),
        ],
        compiler_params=pltpu.CompilerParams(
            dimension_semantics=("arbitrary",)),
    )(adj, support, t2, bgc2, W00, b002, W10, b102,
      w01, b012, w11, b112, wpp, bpp2)

    return y.reshape(-1), rep, p1.reshape(-1)


# speedup vs baseline: 1.0487x; 1.0242x over previous
import jax
import jax.numpy as jnp
from jax.experimental import pallas as pl
from jax.experimental.pallas import tpu as pltpu


def _support_kernel(x_ref, w_ref, o_ref):
    o_ref[...] = jnp.dot(x_ref[...], w_ref[...],
                         preferred_element_type=jnp.float32
                         ).astype(jnp.bfloat16)


def _main_kernel(adj_ref, s_ref, t_ref, bgc_ref, W00_ref, b00_ref,
                 W10_ref, b10_ref, w01_ref, b01_ref, w11_ref, b11_ref,
                 wpp_ref, bpp_ref, rep_ref, y_ref, p1_ref):
    acc = jnp.dot(adj_ref[...].astype(jnp.bfloat16), s_ref[...],
                  preferred_element_type=jnp.float32)
    rep = jnp.maximum(acc + bgc_ref[...], 0.0)
    rep_ref[...] = rep
    y00 = jnp.maximum(
        jnp.dot(rep, W00_ref[...], preferred_element_type=jnp.float32)
        + b00_ref[...], 0.0)
    y10 = jnp.maximum(
        jnp.dot(rep, W10_ref[...], preferred_element_type=jnp.float32)
        + b10_ref[...], 0.0)
    y0 = jnp.dot(y00, w01_ref[...],
                 preferred_element_type=jnp.float32) + b01_ref[...]
    y1 = jnp.dot(y10, w11_ref[...],
                 preferred_element_type=jnp.float32) + b11_ref[...]
    y_ref[...] = jnp.where(t_ref[...] > 0, y1, y0)
    p1_ref[...] = jax.nn.sigmoid(
        jnp.dot(rep, wpp_ref[...], preferred_element_type=jnp.float32)
        + bpp_ref[...])


def kernel(x, adj, t, W_gc, b_gc, W00, b00, W10, b10, w01, b01, w11,
           b11, wpp, bpp):
    N, F = x.shape
    H = W_gc.shape[1]
    BM = 400

    support = pl.pallas_call(
        _support_kernel,
        out_shape=jax.ShapeDtypeStruct((N, H), jnp.bfloat16),
    )(x, W_gc)

    t2 = t.reshape(N, 1)
    bgc2 = b_gc.reshape(1, H)
    b002 = b00.reshape(1, H)
    b102 = b10.reshape(1, H)
    b012 = b01.reshape(1, 1)
    b112 = b11.reshape(1, 1)
    bpp2 = bpp.reshape(1, 1)

    full = lambda shape: pl.BlockSpec(shape, lambda i: (0, 0))
    grid = (N // BM,)
    rep, y, p1 = pl.pallas_call(
        _main_kernel,
        grid=grid,
        in_specs=[
            pl.BlockSpec((BM, N), lambda i: (i, 0)),
            full((N, H)),
            pl.BlockSpec((BM, 1), lambda i: (i, 0)),
            full((1, H)),
            full((H, H)),
            full((1, H)),
            full((H, H)),
            full((1, H)),
            full((H, 1)),
            full((1, 1)),
            full((H, 1)),
            full((1, 1)),
            full((H, 1)),
            full((1, 1)),
        ],
        out_specs=[
            pl.BlockSpec((BM, H), lambda i: (i, 0)),
            pl.BlockSpec((BM, 1), lambda i: (i, 0)),
            pl.BlockSpec((BM, 1), lambda i: (i, 0)),
        ],
        out_shape=[
            jax.ShapeDtypeStruct((N, H), jnp.float32),
            jax.ShapeDtypeStruct((N, 1), jnp.float32),
            jax.ShapeDtypeStruct((N, 1), jnp.float32),
        ],
        compiler_params=pltpu.CompilerParams(
            dimension_semantics=("arbitrary",)),
    )(adj, support, t2, bgc2, W00, b002, W10, b102,
      w01, b012, w11, b112, wpp, bpp2)

    return y.reshape(-1), rep, p1.reshape(-1)


# speedup vs baseline: 1.0773x; 1.0273x over previous
import jax
import jax.numpy as jnp
from jax.experimental import pallas as pl
from jax.experimental.pallas import tpu as pltpu

_BM = 200
_NBUF = 4


def _fused_kernel(x_ref, Wgc_ref, adj_ref, t_ref, bgc_ref, W00_ref,
                  b00_ref, W10_ref, b10_ref, w01_ref, b01_ref, w11_ref,
                  b11_ref, wpp_ref, bpp_ref,
                  rep_ref, y_ref, p1_ref,
                  sup_ref, buf_ref, sem_ref):
    n = adj_ref.shape[0]
    nsteps = n // _BM

    for b in range(_NBUF):
        pltpu.make_async_copy(adj_ref.at[pl.ds(b * _BM, _BM), :],
                              buf_ref.at[b], sem_ref.at[b]).start()

    sup_ref[...] = jnp.dot(x_ref[...], Wgc_ref[...],
                           preferred_element_type=jnp.float32
                           ).astype(jnp.bfloat16)

    def body(s, carry):
        b = jax.lax.rem(s, _NBUF)
        row = s * _BM
        pltpu.make_async_copy(adj_ref.at[pl.ds(row, _BM), :],
                              buf_ref.at[b], sem_ref.at[b]).wait()
        acc = jnp.dot(buf_ref[b].astype(jnp.bfloat16), sup_ref[...],
                      preferred_element_type=jnp.float32)
        rep = jnp.maximum(acc + bgc_ref[...], 0.0)
        rep_ref[pl.ds(row, _BM), :] = rep
        y00 = jnp.maximum(
            jnp.dot(rep, W00_ref[...], preferred_element_type=jnp.float32)
            + b00_ref[...], 0.0)
        y10 = jnp.maximum(
            jnp.dot(rep, W10_ref[...], preferred_element_type=jnp.float32)
            + b10_ref[...], 0.0)
        y0 = jnp.dot(y00, w01_ref[...],
                     preferred_element_type=jnp.float32) + b01_ref[...]
        y1 = jnp.dot(y10, w11_ref[...],
                     preferred_element_type=jnp.float32) + b11_ref[...]
        y_ref[pl.ds(row, _BM), :] = jnp.where(
            t_ref[pl.ds(row, _BM), :] > 0, y1, y0)
        p1_ref[pl.ds(row, _BM), :] = jax.nn.sigmoid(
            jnp.dot(rep, wpp_ref[...], preferred_element_type=jnp.float32)
            + bpp_ref[...])

        nxt = s + _NBUF

        @pl.when(nxt < nsteps)
        def _():
            pltpu.make_async_copy(adj_ref.at[pl.ds(nxt * _BM, _BM), :],
                                  buf_ref.at[b], sem_ref.at[b]).start()
        return carry

    jax.lax.fori_loop(0, nsteps, body, 0)


def kernel(x, adj, t, W_gc, b_gc, W00, b00, W10, b10, w01, b01, w11,
           b11, wpp, bpp):
    N, F = x.shape
    H = W_gc.shape[1]

    t2 = t.reshape(N, 1)
    bgc2 = b_gc.reshape(1, H)
    b002 = b00.reshape(1, H)
    b102 = b10.reshape(1, H)
    b012 = b01.reshape(1, 1)
    b112 = b11.reshape(1, 1)
    bpp2 = bpp.reshape(1, 1)

    vmem = pl.BlockSpec(memory_space=pltpu.VMEM)
    rep, y, p1 = pl.pallas_call(
        _fused_kernel,
        in_specs=[vmem, vmem,
                  pl.BlockSpec(memory_space=pl.ANY),
                  vmem, vmem, vmem, vmem, vmem, vmem, vmem, vmem, vmem,
                  vmem, vmem, vmem],
        out_specs=[vmem, vmem, vmem],
        out_shape=[
            jax.ShapeDtypeStruct((N, H), jnp.float32),
            jax.ShapeDtypeStruct((N, 1), jnp.float32),
            jax.ShapeDtypeStruct((N, 1), jnp.float32),
        ],
        scratch_shapes=[
            pltpu.VMEM((N, H), jnp.bfloat16),
            pltpu.VMEM((_NBUF, _BM, N), jnp.float32),
            pltpu.SemaphoreType.DMA((_NBUF,)),
        ],
    )(x, W_gc, adj, t2, bgc2, W00, b002, W10, b102,
      w01, b012, w11, b112, wpp, bpp2)

    return y.reshape(-1), rep, p1.reshape(-1)
